# baseline (speedup 1.0000x reference)
import functools

import jax
import jax.numpy as jnp
from jax import lax
from jax.experimental import pallas as pl
from jax.experimental.pallas import tpu as pltpu
from jax.experimental.pallas import tpu_sc as plsc

_D = 64
_NW = 32
_CHUNK = 128
_SCALE = 8.0


def _emb_body(idx_hbm, table_hbm, out_hbm, idx_v, rows_v, sem):
    n_chunks = idx_v.shape[0]
    wid = lax.axis_index("s") * 2 + lax.axis_index("c")
    base = wid * (n_chunks * _CHUNK)
    pltpu.sync_copy(idx_hbm.at[wid], idx_v)

    def chunk_body(j, carry):
        pltpu.async_copy(table_hbm.at[idx_v.at[j]], rows_v, sem).wait()

        def scale_body(i, c):
            for sub in range(_D // 16):
                s = pl.ds(sub * 16, 16)
                rows_v[i, s] = rows_v[i, s] * _SCALE
            return c

        lax.fori_loop(0, _CHUNK, scale_body, 0, unroll=2)

        pltpu.sync_copy(rows_v, out_hbm.at[pl.ds(base + j * _CHUNK, _CHUNK)])
        return carry

    lax.fori_loop(0, n_chunks, chunk_body, 0)


def kernel(x, emb_weight):
    b0, b1 = x.shape
    total = b0 * b1
    n_chunks = total // (_NW * _CHUNK)
    idx = x.reshape(_NW, n_chunks, _CHUNK).astype(jnp.int32)

    mesh = plsc.VectorSubcoreMesh(core_axis_name="c", subcore_axis_name="s")
    run = functools.partial(
        pl.kernel,
        out_type=jax.ShapeDtypeStruct((total, _D), jnp.float32),
        mesh=mesh,
        scratch_types=[
            pltpu.VMEM((n_chunks, _CHUNK), jnp.int32),
            pltpu.VMEM((_CHUNK, _D), jnp.float32),
            pltpu.SemaphoreType.DMA,
        ],
        compiler_params=pltpu.CompilerParams(use_tc_tiling_on_sc=False),
    )(_emb_body)
    out = run(idx, emb_weight)
    return out.reshape(b0, b1, _D)


# speedup vs baseline: 1.0548x; 1.0548x over previous
import functools

import jax
import jax.numpy as jnp
from jax import lax
from jax.experimental import pallas as pl
from jax.experimental.pallas import tpu as pltpu
from jax.experimental.pallas import tpu_sc as plsc

_D = 64
_NW = 32
_CHUNK = 128
_NB = 4
_SCALE = 8.0


def _emb_body(idx_hbm, table_hbm, out_hbm, idx_v, bufg, bufo, *sems):
    n_chunks = idx_v.shape[0]
    n_groups = n_chunks // _NB
    sem_g, sem_o = sems[:_NB], sems[_NB:]
    wid = lax.axis_index("s") * 2 + lax.axis_index("c")
    base = wid * (n_chunks * _CHUNK)
    pltpu.sync_copy(idx_hbm.at[wid], idx_v)

    def gather_start(j, b):
        pltpu.async_copy(table_hbm.at[idx_v.at[j]], bufg.at[b], sem_g[b])

    def gather_wait(j, b):
        pltpu.make_async_copy(table_hbm.at[idx_v.at[j]], bufg.at[b],
                              sem_g[b]).wait()

    def out_slot(j):
        return out_hbm.at[pl.ds(base + j * _CHUNK, _CHUNK)]

    def out_start(j, b):
        pltpu.async_copy(bufo.at[b], out_slot(j), sem_o[b])

    def out_wait(j, b):
        pltpu.make_async_copy(bufo.at[b], out_slot(j), sem_o[b]).wait()

    def scale_chunk(b):
        src, dst = bufg.at[b], bufo.at[b]

        def row(i, c):
            for sub in range(_D // 16):
                s = pl.ds(sub * 16, 16)
                dst[i, s] = src[i, s] * _SCALE
            return c

        lax.fori_loop(0, _CHUNK, row, 0, unroll=4)

    def group(g, first, fire):
        for b in range(_NB):
            j = g * _NB + b
            gather_wait(j, b)
            if not first:
                out_wait(j, b)
            scale_chunk(b)
            if fire:
                gather_start(j + _NB, b)
            out_start(j, b)

    for b in range(_NB):
        gather_start(b, b)
    group(0, first=True, fire=True)
    lax.fori_loop(1, n_groups - 1,
                  lambda g, c: (group(g, first=False, fire=True), c)[1], 0)
    group(n_groups - 1, first=False, fire=False)
    for b in range(_NB):
        out_wait((n_groups - 1) * _NB + b, b)


def kernel(x, emb_weight):
    b0, b1 = x.shape
    total = b0 * b1
    n_chunks = total // (_NW * _CHUNK)
    idx = x.reshape(_NW, n_chunks, _CHUNK).astype(jnp.int32)

    mesh = plsc.VectorSubcoreMesh(core_axis_name="c", subcore_axis_name="s")
    run = functools.partial(
        pl.kernel,
        out_type=jax.ShapeDtypeStruct((total, _D), jnp.float32),
        mesh=mesh,
        scratch_types=[
            pltpu.VMEM((n_chunks, _CHUNK), jnp.int32),
            pltpu.VMEM((_NB, _CHUNK, _D), jnp.float32),
            pltpu.VMEM((_NB, _CHUNK, _D), jnp.float32),
        ] + [pltpu.SemaphoreType.DMA] * (2 * _NB),
        compiler_params=pltpu.CompilerParams(use_tc_tiling_on_sc=False),
    )(_emb_body)
    out = run(idx, emb_weight)
    return out.reshape(b0, b1, _D)
